# baseline (speedup 1.0000x reference)
import functools
import jax
import jax.numpy as jnp
from jax import lax
from jax.experimental import pallas as pl
from jax.experimental.pallas import tpu as pltpu
from jax.experimental.pallas import tpu_sc as plsc

_N = 262144
_NS = 16
_CHUNK = _N // _NS
_L = 16
_U = 8


_GATHER_DNUMS = lax.GatherDimensionNumbers(
    offset_dims=(), collapsed_slice_dims=(0,), start_index_map=(0,)
)


def _lane_shuffle(v, idx):
    return lax.gather(
        v,
        idx[:, None],
        dimension_numbers=_GATHER_DNUMS,
        slice_sizes=(1,),
        mode=lax.GatherScatterMode.PROMISE_IN_BOUNDS,
    )


def _xlane(v, op):
    lanes = lax.iota(jnp.int32, _L)
    for shift in (1, 2, 4, 8):
        v = op(v, _lane_shuffle(v, lanes ^ shift))
    return v


def _make_sc_kernel():
    mesh = plsc.VectorSubcoreMesh(
        core_axis_name="c", subcore_axis_name="s", num_cores=1
    )

    @functools.partial(
        pl.kernel,
        mesh=mesh,
        compiler_params=pltpu.CompilerParams(needs_layout_passes=False),
        out_type=(
            jax.ShapeDtypeStruct((_L,), jnp.float32),
            jax.ShapeDtypeStruct((_NS, _L), jnp.float32),
        ),
        scratch_types=[
            pltpu.VMEM((_CHUNK,), jnp.float32),
            pltpu.VMEM((_CHUNK,), jnp.float32),
            pltpu.VMEM((_CHUNK,), jnp.float32),
            pltpu.VMEM((_L,), jnp.float32),
            pltpu.VMEM((_L,), jnp.float32),
            pltpu.VMEM((_NS, _L), jnp.float32),
            pltpu.VMEM((_L,), jnp.float32),
        ],
    )
    def ghm_sc(pred_hbm, target_hbm, dens_hbm, out_hbm, shared,
               p_v, t_v, g_v, tab_v, row_v, all_v, out_v):
        sid = lax.axis_index("s")
        base = sid * _CHUNK
        pltpu.sync_copy(pred_hbm.at[pl.ds(base, _CHUNK)], p_v)
        pltpu.sync_copy(target_hbm.at[pl.ds(base, _CHUNK)], t_v)
        pltpu.sync_copy(dens_hbm, tab_v)

        def body1(i, carry):
            ms = list(carry)
            for u in range(_U):
                off = (i * _U + u) * _L
                pv = p_v[pl.ds(off, _L)]
                tv = t_v[pl.ds(off, _L)]
                g = jnp.abs(pv - tv)
                g_v[pl.ds(off, _L)] = g
                ms[u] = jnp.maximum(ms[u], g)
            return tuple(ms)

        zeros = jnp.zeros((_L,), jnp.float32)
        ms = lax.fori_loop(0, _CHUNK // (_L * _U), body1, (zeros,) * _U)
        m = ms[0]
        for u in range(1, _U):
            m = jnp.maximum(m, ms[u])

        row_v[...] = m
        pltpu.sync_copy(row_v, shared.at[sid])
        plsc.subcore_barrier()
        pltpu.sync_copy(shared, all_v)
        m2 = zeros
        for r in range(_NS):
            m2 = jnp.maximum(m2, all_v[r, :])
        gmax = _xlane(m2, jnp.maximum)
        plsc.subcore_barrier()

        tab_v[...] = 1.0 / (tab_v[...] + 1e-6)

        nine = jnp.float32(9.0)

        def body2(i, carry):
            accs = list(carry)
            for u in range(_U):
                off = (i * _U + u) * _L
                g = g_v[pl.ds(off, _L)]
                scaled = g / gmax * nine
                idx = jnp.clip(scaled.astype(jnp.int32), 0, 9)
                w = plsc.load_gather(tab_v, [idx])
                accs[u] = accs[u] + w * g * g
            return tuple(accs)

        accs = lax.fori_loop(0, _CHUNK // (_L * _U), body2, (zeros,) * _U)
        acc = accs[0]
        for u in range(1, _U):
            acc = acc + accs[u]

        row_v[...] = acc
        pltpu.sync_copy(row_v, shared.at[sid])
        plsc.subcore_barrier()

        @pl.when(sid == 0)
        def _():
            pltpu.sync_copy(shared, all_v)
            tot = jnp.zeros((_L,), jnp.float32)
            for r in range(_NS):
                tot = tot + all_v[r, :]
            tot = _xlane(tot, jnp.add)
            out_v[...] = tot * (1.0 / _N)
            pltpu.sync_copy(out_v, out_hbm)

    return ghm_sc


_GHM_SC = _make_sc_kernel()


def kernel(pred, target, gradient_hist, grad_density):
    del gradient_hist
    dens16 = jnp.pad(grad_density, (0, _L - grad_density.shape[0]))
    out, _ = _GHM_SC(pred, target, dens16)
    return out[0]


# speedup vs baseline: 3.3070x; 3.3070x over previous
import jax
import jax.numpy as jnp
from jax.experimental import pallas as pl

_N = 262144
_ROWS = 512
_COLS = 512


def _ghm_kernel(pred_ref, target_ref, dens_ref, out_ref):
    p = pred_ref[...]
    t = target_ref[...]
    diff = p - t
    g = jnp.abs(diff)
    gmax = jnp.max(g)
    bins = dens_ref.shape[-1]
    scaled = g / gmax * (bins - 1)
    idx = jnp.clip(scaled.astype(jnp.int32), 0, bins - 1)
    w = jnp.zeros_like(g)
    for b in range(bins):
        wb = 1.0 / (dens_ref[0, b] + 1e-6)
        w = jnp.where(idx == b, wb, w)
    loss = jnp.sum(w * diff * diff) * (1.0 / _N)
    out_ref[...] = jnp.full((1, 1), loss, dtype=jnp.float32)


def kernel(pred, target, gradient_hist, grad_density):
    del gradient_hist
    p2 = pred.reshape(_ROWS, _COLS)
    t2 = target.reshape(_ROWS, _COLS)
    d2 = grad_density.reshape(1, -1)
    out = pl.pallas_call(
        _ghm_kernel,
        out_shape=jax.ShapeDtypeStruct((1, 1), jnp.float32),
    )(p2, t2, d2)
    return out[0, 0]
